# baseline (speedup 1.0000x reference)
import functools
import math

import jax
import jax.numpy as jnp
from jax import lax
from jax.experimental import pallas as pl
from jax.experimental.pallas import tpu as pltpu
from jax.experimental.pallas import tpu_sc as plsc

NC, NS, L = 2, 16, 16
NW = NC * NS


def _make_sc_kernel(B, C, HW, PAIRS, PAD):
    PPW = PAD // NW
    CCH = C // L

    mesh = plsc.VectorSubcoreMesh(core_axis_name="c", subcore_axis_name="s")

    @functools.partial(
        pl.kernel,
        out_type=jax.ShapeDtypeStruct((NW, 2, L), jnp.float32),
        mesh=mesh,
        compiler_params=pltpu.CompilerParams(needs_layout_passes=False),
        scratch_types=[
            pltpu.VMEM((PPW,), jnp.int32),
            pltpu.VMEM((PPW,), jnp.float32),
            pltpu.VMEM((PPW,), jnp.int32),
            pltpu.VMEM((PPW, C), jnp.int32),
            pltpu.VMEM((PPW, C), jnp.float32),
            pltpu.VMEM((PPW, C), jnp.float32),
            pltpu.VMEM((2, L), jnp.float32),
            pltpu.SemaphoreType.DMA,
            pltpu.SemaphoreType.DMA,
        ],
    )
    def sc_kernel(flat_hbm, base_hbm, maskf_hbm, rows_hbm, tgt_hbm, out_hbm,
                  base_v, maskf_v, rows_v, idx_v, gath_v, tgt_v, part_v,
                  sem0, sem1):
        wid = lax.axis_index("s") * NC + lax.axis_index("c")

        pltpu.sync_copy(base_hbm.at[wid], base_v)
        pltpu.sync_copy(maskf_hbm.at[wid], maskf_v)
        pltpu.sync_copy(rows_hbm.at[wid], rows_v)

        tgt_cp = pltpu.async_copy(tgt_hbm.at[rows_v], tgt_v, sem1)

        coffs = [(jnp.arange(L, dtype=jnp.int32) + cc * L) * HW
                 for cc in range(CCH)]

        def build_body(j, carry):
            bsplat = plsc.load_gather(base_v, [jnp.full((L,), j, jnp.int32)])
            for cc in range(CCH):
                idx_v[j, pl.ds(cc * L, L)] = bsplat + coffs[cc]
            return carry

        lax.fori_loop(0, PPW, build_body, 0)

        def fire_body(j, carry):
            pltpu.make_async_copy(
                flat_hbm.at[idx_v.at[j]], gath_v.at[j], sem0).start()
            return carry

        lax.fori_loop(0, PPW, fire_body, 0)
        pltpu.make_async_copy(tgt_hbm.at[pl.ds(0, PPW)], gath_v, sem0).wait()
        tgt_cp.wait()

        def mse_body(j, acc):
            mf = plsc.load_gather(maskf_v, [jnp.full((L,), j, jnp.int32)])
            for cc in range(CCH):
                d = gath_v[j, pl.ds(cc * L, L)] - tgt_v[j, pl.ds(cc * L, L)]
                acc = acc + d * d * mf
            return acc

        acc = lax.fori_loop(0, PPW, mse_body, jnp.zeros((L,), jnp.float32))

        cnt = jnp.zeros((L,), jnp.float32)
        for jj in range(PPW // L):
            cnt = cnt + maskf_v[pl.ds(jj * L, L)]

        part_v[0, :] = acc
        part_v[1, :] = cnt
        pltpu.sync_copy(part_v, out_hbm.at[wid])

    return sc_kernel


def kernel(output, mask, ind, target):
    B, C, H, W = output.shape
    K = ind.shape[1]
    HW = H * W
    PAIRS = B * K
    PAD = ((PAIRS + 8 * NW - 1) // (8 * NW)) * (8 * NW)

    flat = output.reshape(-1)
    tgt2d = target.reshape(PAIRS, C)

    p = jnp.arange(PAD, dtype=jnp.int32)
    valid = p < PAIRS
    psafe = jnp.minimum(p, PAIRS - 1)
    ind_flat = ind.reshape(-1).astype(jnp.int32)[psafe]
    b_of_p = psafe // K
    base = jnp.where(valid, b_of_p * (C * HW) + ind_flat, 0).reshape(NW, -1)
    maskf = (mask.reshape(-1) > 0).astype(jnp.float32)[psafe]
    maskf = jnp.where(valid, maskf, 0.0).reshape(NW, -1)
    rows = psafe.reshape(NW, -1)

    sck = _make_sc_kernel(B, C, HW, PAIRS, PAD)
    parts = sck(flat, base, maskf, rows, tgt2d)

    sumsq = jnp.sum(parts[:, 0, :])
    cnt = jnp.sum(parts[:, 1, :])
    denom = jnp.maximum(cnt * C, 1.0)
    return jnp.where(cnt > 0, sumsq / denom, jnp.asarray(0.0, jnp.float32))


# speedup vs baseline: 1.3855x; 1.3855x over previous
import functools
import math

import jax
import jax.numpy as jnp
from jax import lax
from jax.experimental import pallas as pl
from jax.experimental.pallas import tpu as pltpu
from jax.experimental.pallas import tpu_sc as plsc

NC, NS, L = 2, 16, 16
NW = NC * NS
TL = 128


def _make_sc_kernel(B, C, H, W, K, PAIRS, PAD):
    PPW = PAD // NW
    CCH = C // L

    mesh = plsc.VectorSubcoreMesh(core_axis_name="c", subcore_axis_name="s")

    @functools.partial(
        pl.kernel,
        out_type=jax.ShapeDtypeStruct((NW, 2, L), jnp.float32),
        mesh=mesh,
        compiler_params=pltpu.CompilerParams(needs_layout_passes=False),
        scratch_types=[
            pltpu.VMEM((PPW,), jnp.int32),
            pltpu.VMEM((PPW,), jnp.int32),
            pltpu.VMEM((PPW,), jnp.int32),
            pltpu.VMEM((PPW,), jnp.float32),
            pltpu.VMEM((PPW,), jnp.int32),
            pltpu.VMEM((2, C), jnp.int32),
            pltpu.VMEM((2, C, TL), jnp.float32),
            pltpu.VMEM((PPW, C), jnp.float32),
            pltpu.VMEM((PPW, C), jnp.float32),
            pltpu.VMEM((2, L), jnp.float32),
            pltpu.SemaphoreType.DMA,
            pltpu.SemaphoreType.DMA,
        ],
    )
    def sc_kernel(out4d_hbm, rb_hbm, wt_hbm, wm_hbm, maskf_hbm, rows_hbm,
                  tgt_hbm, out_hbm, rb_v, wt_v, wm_v, maskf_v, rows_v,
                  idx_v, win_v, gath_v, tgt_v, part_v, sem0, sem1):
        rowmap_hbm = out4d_hbm.reshape(B * C * H, W)
        wid = lax.axis_index("s") * NC + lax.axis_index("c")
        lane = jnp.arange(L, dtype=jnp.int32)

        pltpu.sync_copy(rb_hbm.at[wid], rb_v)
        pltpu.sync_copy(wt_hbm.at[wid], wt_v)
        pltpu.sync_copy(wm_hbm.at[wid], wm_v)
        pltpu.sync_copy(maskf_hbm.at[wid], maskf_v)
        pltpu.sync_copy(rows_hbm.at[wid], rows_v)

        tgt_cp = pltpu.async_copy(tgt_hbm.at[rows_v], tgt_v, sem1)

        def _scalar_at(ref, j):
            chunk = ref[pl.ds((j // L) * L, L)]
            sel = jnp.where(lane == j % L, chunk, 0)
            return lax.reduce_sum_p.bind(sel, axes=(0,))

        coffs = [(lane + cc * L) * H for cc in range(CCH)]

        def _fire(j):
            par = j % 2
            rbs = plsc.load_gather(rb_v, [jnp.full((L,), j, jnp.int32)])
            for cc in range(CCH):
                idx_v[par, pl.ds(cc * L, L)] = rbs + coffs[cc]
            wt_s = pl.multiple_of(_scalar_at(wt_v, j) * TL, TL)
            pltpu.make_async_copy(
                rowmap_hbm.at[idx_v.at[par], pl.ds(wt_s, TL)],
                win_v.at[par], sem0).start()

        def _wait_one():
            pltpu.make_async_copy(
                rowmap_hbm.at[pl.ds(0, C), pl.ds(0, TL)],
                win_v.at[0], sem0).wait()

        def _extract(j):
            par = j % 2
            wmf = jnp.full((L,), 1, jnp.int32) * _scalar_at(wm_v, j)
            pars = jnp.full((L,), par, jnp.int32)
            for cc in range(CCH):
                gath_v[j, pl.ds(cc * L, L)] = plsc.load_gather(
                    win_v, [pars, cc * L + lane, wmf])

        def pipe_body(j, carry):
            _wait_one()
            _extract(j)

            @pl.when(j + 2 < PPW)
            def _():
                _fire(j + 2)

            return carry

        _fire(0)
        _fire(1)
        lax.fori_loop(0, PPW, pipe_body, 0)
        tgt_cp.wait()

        def mse_body(j, acc):
            mf = plsc.load_gather(maskf_v, [jnp.full((L,), j, jnp.int32)])
            for cc in range(CCH):
                d = gath_v[j, pl.ds(cc * L, L)] - tgt_v[j, pl.ds(cc * L, L)]
                acc = acc + d * d * mf
            return acc

        acc = lax.fori_loop(0, PPW, mse_body, jnp.zeros((L,), jnp.float32))

        cnt = jnp.zeros((L,), jnp.float32)
        for jj in range(PPW // L):
            cnt = cnt + maskf_v[pl.ds(jj * L, L)]

        part_v[0, :] = acc
        part_v[1, :] = cnt
        pltpu.sync_copy(part_v, out_hbm.at[wid])

    return sc_kernel


def kernel(output, mask, ind, target):
    B, C, H, W = output.shape
    K = ind.shape[1]
    PAIRS = B * K
    PAD = ((PAIRS + 8 * NW - 1) // (8 * NW)) * (8 * NW)

    tgt2d = target.reshape(PAIRS, C)

    p = jnp.arange(PAD, dtype=jnp.int32)
    valid = p < PAIRS
    psafe = jnp.minimum(p, PAIRS - 1)
    ind_flat = ind.reshape(-1).astype(jnp.int32)[psafe]
    hh = jnp.where(valid, ind_flat // W, 0)
    ww = jnp.where(valid, ind_flat % W, 0)
    b_of_p = jnp.minimum(psafe // K, B - 1)
    rb = jnp.where(valid, b_of_p * (C * H) + hh, 0).reshape(NW, -1)
    wt = (ww // TL).reshape(NW, -1)
    wm = (ww % TL).reshape(NW, -1)
    maskf = (mask.reshape(-1) > 0).astype(jnp.float32)[psafe]
    maskf = jnp.where(valid, maskf, 0.0).reshape(NW, -1)
    rows = psafe.reshape(NW, -1)

    sck = _make_sc_kernel(B, C, H, W, K, PAIRS, PAD)
    parts = sck(output, rb, wt, wm, maskf, rows, tgt2d)

    sumsq = jnp.sum(parts[:, 0, :])
    cnt = jnp.sum(parts[:, 1, :])
    denom = jnp.maximum(cnt * C, 1.0)
    return jnp.where(cnt > 0, sumsq / denom, jnp.asarray(0.0, jnp.float32))
